# baseline (speedup 1.0000x reference)
import functools

import jax
import jax.numpy as jnp
import numpy as np
from jax import lax
from jax.experimental import pallas as pl
from jax.experimental.pallas import tpu as pltpu
from jax.experimental.pallas import tpu_sc as plsc

B, C, L, D = 16384, 26, 128, 64
TAU = 0.07
MARGIN = 0.3


BB = 512
NB = B // BB

_OH = np.kron(np.eye(C, dtype=np.float32), np.ones((1, BB), np.float32))


def _tc_body(logT_ref, labT_ref, tft_ref, pro_ref, oh_ref, out_ref):
    x = logT_ref[...]
    y = labT_ref[...]

    bce = jnp.maximum(x, 0.0) - x * y + jnp.log1p(jnp.exp(-jnp.abs(x)))
    acc = jnp.sum(bce, axis=0)

    p = pro_ref[...]
    pinv = 1.0 / jnp.maximum(
        jnp.sqrt(jnp.sum(p * p, axis=1, keepdims=True)), 1e-12)
    pn = p * pinv
    f = tft_ref[...]
    F = f.reshape(C * BB, D)
    ones_row = jnp.ones((1, D), jnp.float32)
    sqv = jax.lax.dot_general(ones_row, F * F, (((1,), (1,)), ((), ())),
                              preferred_element_type=jnp.float32)
    finv = 1.0 / jnp.maximum(jnp.sqrt(sqv), 1e-12)
    St = jax.lax.dot_general(pn, F, (((1,), (1,)), ((), ())),
                             preferred_element_type=jnp.float32)
    St = St * (finv * (1.0 / TAU))
    m = jnp.max(St, axis=0, keepdims=True)
    lse = m + jnp.log(jnp.sum(jnp.exp(St - m), axis=0, keepdims=True))
    diag = jnp.sum(St * oh_ref[...], axis=0, keepdims=True)
    psc = (lse - diag).reshape(C, BB)
    acc = acc + jnp.sum(jnp.where(y > 0.0, psc, 0.0), axis=0)

    out_ref[...] = acc


def _tc_call(logT, labT, tft, pro, oh):
    return pl.pallas_call(
        _tc_body,
        grid=(NB,),
        in_specs=[
            pl.BlockSpec((C, BB), lambda i: (0, i)),
            pl.BlockSpec((C, BB), lambda i: (0, i)),
            pl.BlockSpec((C, BB, D), lambda i: (0, i, 0)),
            pl.BlockSpec((C, D), lambda i: (0, 0)),
            pl.BlockSpec((C, C * BB), lambda i: (0, 0)),
        ],
        out_specs=pl.BlockSpec((BB,), lambda i: (i,)),
        out_shape=jax.ShapeDtypeStruct((B,), jnp.float32),
    )(logT, labT, tft, pro, oh)


_NC = 2
_NS = 16
_NW = _NC * _NS
_SPW = B // _NW
_RPW = C * _SPW
_GCH = 64
_PAD = 4 * _GCH
_NEG = -1e30


def _sc_body(lab_hbm, tlt_hbm, out_hbm, lab_v, bufa, bufb, acc_v, sema, semb):
    wid = lax.axis_index("s") * _NC + lax.axis_index("c")
    wbase = wid * _SPW

    iota16 = lax.iota(jnp.int32, 16)

    def _initacc(i, _):
        acc_v[pl.ds(i * 16, 16)] = jnp.zeros((16,), jnp.float32)
        return 0
    lax.fori_loop(0, _SPW // 16, _initacc, 0)

    pltpu.sync_copy(lab_hbm.at[pl.ds(wbase * C, _SPW * C)], lab_v)

    NCH = _SPW // _GCH
    NK = C * NCH

    def _start(k, buf, sem):
        c = k // NCH
        ch = k % NCH
        base = jnp.minimum(c * B + wbase + ch * _GCH, C * B - _GCH)
        pltpu.async_copy(tlt_hbm.at[pl.ds(base, _GCH)], buf, sem)

    def _wait(buf, sem):
        pltpu.make_async_copy(tlt_hbm.at[pl.ds(0, _GCH)], buf, sem).wait()

    def _consume(k, buf):
        c = k // NCH
        ch = k % NCH
        for g in range(_GCH // 16):
            boff = ch * _GCH + g * 16
            ridx = g * 16 + iota16
            zz = iota16 - iota16
            accs = [jnp.full((16,), _NEG, jnp.float32) for _ in range(4)]
            for l in range(L):
                accs[l % 4] = jnp.maximum(
                    accs[l % 4], plsc.load_gather(buf, [ridx, zz + l]))
            acc16 = jnp.maximum(jnp.maximum(accs[0], accs[1]),
                                jnp.maximum(accs[2], accs[3]))
            rej = 1.0 / (1.0 + jnp.exp(-acc16)) - MARGIN
            rej = jnp.maximum(rej, 0.0)
            lv = plsc.load_gather(lab_v, [(boff + iota16) * C + c])
            rej = jnp.where(lv < 0.5, rej, 0.0)
            cur = acc_v[pl.ds(boff, 16)]
            acc_v[pl.ds(boff, 16)] = cur + rej

    _start(0, bufa, sema)
    _start(1, bufb, semb)

    def _gloop(jj, _):
        k0 = jj * 2
        _wait(bufa, sema)
        _start(k0 + 2, bufa, sema)
        _consume(k0, bufa)
        _wait(bufb, semb)
        _start(k0 + 3, bufb, semb)
        _consume(k0 + 1, bufb)
        return 0
    lax.fori_loop(0, NK // 2, _gloop, 0)

    _wait(bufa, sema)
    _wait(bufb, semb)

    pltpu.sync_copy(acc_v.at[pl.ds(0, _SPW)], out_hbm.at[pl.ds(wbase, _SPW)])


_SC_CACHE = []


def _get_sc_rejection():
    if not _SC_CACHE:
        k = pl.kernel(
            _sc_body,
            mesh=plsc.VectorSubcoreMesh(core_axis_name="c",
                                        subcore_axis_name="s"),
            out_type=jax.ShapeDtypeStruct((B,), jnp.float32),
            compiler_params=pltpu.CompilerParams(needs_layout_passes=False),
            scratch_types=[
                pltpu.VMEM((_SPW * C,), jnp.float32),
                pltpu.VMEM((_GCH, L), jnp.float32),
                pltpu.VMEM((_GCH, L), jnp.float32),
                pltpu.VMEM((_SPW,), jnp.float32),
                pltpu.SemaphoreType.DMA,
                pltpu.SemaphoreType.DMA,
            ],
        )
        _SC_CACHE.append(k)
    return _SC_CACHE[0]


def kernel(logits, total_cls_logits, total_cls_feature, labels, prototypes):
    logT = logits.T
    labT = labels.T
    oh = jnp.asarray(_OH)
    tc_out = _tc_call(logT, labT, total_cls_feature, prototypes, oh)
    tlt_rows = total_cls_logits.reshape(C * B, L)
    sc_out = _get_sc_rejection()(labels.reshape(B * C), tlt_rows)
    return tc_out + sc_out


# speedup vs baseline: 3.0427x; 3.0427x over previous
import jax
import jax.numpy as jnp
import numpy as np
from jax.experimental import pallas as pl
from jax.experimental.pallas import tpu as pltpu

B, C, L, D = 16384, 26, 128, 64
TAU = 0.07
MARGIN = 0.3

BB = 512
NB = B // BB

_OH = np.kron(np.eye(C, dtype=np.float32), np.ones((1, BB), np.float32))


def _body(logT_ref, labT_ref, tlt_ref, tft_ref, pro_ref, oh_ref, out_ref):
    x = logT_ref[...]
    y = labT_ref[...]

    bce = jnp.maximum(x, 0.0) - x * y + jnp.log1p(jnp.exp(-jnp.abs(x)))
    acc = jnp.sum(bce, axis=0)

    t = tlt_ref[...]
    mx = jnp.max(t, axis=2)
    rej = jnp.maximum(jax.nn.sigmoid(mx) - MARGIN, 0.0)
    acc = acc + jnp.sum(jnp.where(y > 0.0, 0.0, rej), axis=0)

    p = pro_ref[...]
    pinv = 1.0 / jnp.maximum(
        jnp.sqrt(jnp.sum(p * p, axis=1, keepdims=True)), 1e-12)
    pn = p * pinv
    f = tft_ref[...]
    F = f.reshape(C * BB, D)
    ones_row = jnp.ones((1, D), jnp.float32)
    sqv = jax.lax.dot_general(ones_row, F * F, (((1,), (1,)), ((), ())),
                              preferred_element_type=jnp.float32)
    finv = 1.0 / jnp.maximum(jnp.sqrt(sqv), 1e-12)
    St = jax.lax.dot_general(pn, F, (((1,), (1,)), ((), ())),
                             preferred_element_type=jnp.float32)
    St = St * (finv * (1.0 / TAU))
    m = jnp.max(St, axis=0, keepdims=True)
    lse = m + jnp.log(jnp.sum(jnp.exp(St - m), axis=0, keepdims=True))
    diag = jnp.sum(St * oh_ref[...], axis=0, keepdims=True)
    psc = (lse - diag).reshape(C, BB)
    acc = acc + jnp.sum(jnp.where(y > 0.0, psc, 0.0), axis=0)

    out_ref[...] = acc


def kernel(logits, total_cls_logits, total_cls_feature, labels, prototypes):
    logT = logits.T
    labT = labels.T
    oh = jnp.asarray(_OH)
    out = pl.pallas_call(
        _body,
        grid=(NB,),
        in_specs=[
            pl.BlockSpec((C, BB), lambda i: (0, i)),
            pl.BlockSpec((C, BB), lambda i: (0, i)),
            pl.BlockSpec((C, BB, L), lambda i: (0, i, 0)),
            pl.BlockSpec((C, BB, D), lambda i: (0, i, 0)),
            pl.BlockSpec((C, D), lambda i: (0, 0)),
            pl.BlockSpec((C, C * BB), lambda i: (0, 0)),
        ],
        out_specs=pl.BlockSpec((BB,), lambda i: (i,)),
        out_shape=jax.ShapeDtypeStruct((B,), jnp.float32),
    )(logT, labT, total_cls_logits, total_cls_feature, prototypes, oh)
    return out
